# baseline (speedup 1.0000x reference)
import functools

import jax
import jax.numpy as jnp
import numpy as np
from jax import lax
from jax.experimental import pallas as pl
from jax.experimental.pallas import tpu as pltpu
from jax.experimental.pallas import tpu_sc as plsc

EMB = 128; B = 64; L = 256; OUT = 2
NN = 10000

_NW = 32
_CH = 128


@functools.lru_cache(maxsize=None)
def _sc_gather_call(n, D, m_pad):
    cpw = m_pad // (_NW * _CH)
    nbuf = 2 if D > 128 else 4
    mesh = plsc.VectorSubcoreMesh(core_axis_name="c", subcore_axis_name="s")

    @functools.partial(
        pl.kernel, mesh=mesh,
        out_type=jax.ShapeDtypeStruct((m_pad, D), jnp.float32),
        compiler_params=pltpu.CompilerParams(use_tc_tiling_on_sc=False),
        scratch_types=[
            pltpu.VMEM((cpw * _CH,), jnp.int32),
        ] + [pltpu.VMEM((_CH, D), jnp.float32)] * nbuf
          + [pltpu.SemaphoreType.DMA] * nbuf,
    )
    def gather_kernel(table_hbm, idx_hbm, out_hbm, idx_v, *bufsems):
        bufs = bufsems[:nbuf]
        sems = bufsems[nbuf:]
        wid = lax.axis_index("s") * 2 + lax.axis_index("c")
        base = wid * (cpw * _CH)
        pltpu.sync_copy(idx_hbm.at[pl.ds(base, cpw * _CH)], idx_v)

        def start(j, slot):
            pltpu.async_copy(
                table_hbm.at[idx_v.at[pl.ds(j * _CH, _CH)]], bufs[slot], sems[slot])

        def drain(j, slot):
            pltpu.make_async_copy(
                table_hbm.at[idx_v.at[pl.ds(j * _CH, _CH)]], bufs[slot],
                sems[slot]).wait()
            pltpu.sync_copy(bufs[slot], out_hbm.at[pl.ds(base + j * _CH, _CH)])

        for b in range(nbuf):
            if b < cpw:
                start(b, b)

        def loop_body(i, carry):
            j0 = i * nbuf
            for b in range(nbuf):
                @pl.when(j0 + b < cpw)
                def _(b=b):
                    drain(j0 + b, b)

                    @pl.when(j0 + b + nbuf < cpw)
                    def _():
                        start(j0 + b + nbuf, b)
            return carry

        lax.fori_loop(0, (cpw + nbuf - 1) // nbuf, loop_body, 0)

    return gather_kernel


def _pick_dsh(n, Dp, cpw):
    for dsh in sorted({d for d in range(8, Dp + 1, 8) if Dp % d == 0},
                      reverse=True):
        words = n * dsh + 16 * (cpw * _CH + 2 * _CH * dsh)
        if words <= 1_900_000:
            return dsh
    raise ValueError("no feasible shard width")


@functools.lru_cache(maxsize=None)
def _sc_segsum_call(n, Dp, m_pad):
    cpw = m_pad // (_NW * _CH)
    dsh = _pick_dsh(n, Dp, cpw)
    nshard = Dp // dsh
    nr = n // 16
    mesh = plsc.VectorSubcoreMesh(core_axis_name="c", subcore_axis_name="s")

    @functools.partial(
        pl.kernel, mesh=mesh,
        out_type=jax.ShapeDtypeStruct((2, n, Dp), jnp.float32),
        compiler_params=pltpu.CompilerParams(use_tc_tiling_on_sc=False),
        scratch_types=[
            pltpu.VMEM((cpw, _CH), jnp.int32),
            pltpu.VMEM((_CH, dsh), jnp.float32),
            pltpu.VMEM((_CH, dsh), jnp.float32),
            pltpu.VMEM_SHARED((n, dsh), jnp.float32),
            pltpu.SemaphoreType.DMA,
            pltpu.SemaphoreType.DMA,
        ],
    )
    def segsum_kernel(y_hbm, idx_hbm, zero_hbm, out_hbm,
                      idx_v, buf0, buf1, acc, sem0, sem1):
        cid = lax.axis_index("c")
        tid = lax.axis_index("s")
        wid = tid * 2 + cid
        chunk0 = wid * cpw
        pltpu.sync_copy(idx_hbm.at[pl.ds(chunk0, cpw)], idx_v)
        bufs = (buf0, buf1)
        sems = (sem0, sem1)

        for s in range(nshard):
            col = s * dsh
            pltpu.sync_copy(zero_hbm.at[pl.ds(tid * nr, nr)],
                            acc.at[pl.ds(tid * nr, nr)])
            plsc.subcore_barrier()

            def start(j, slot):
                pltpu.async_copy(
                    y_hbm.at[pl.ds((chunk0 + j) * _CH, _CH), pl.ds(col, dsh)],
                    bufs[slot], sems[slot])

            def scat(j, slot):
                pltpu.make_async_copy(
                    y_hbm.at[pl.ds((chunk0 + j) * _CH, _CH), pl.ds(col, dsh)],
                    bufs[slot], sems[slot]).wait()
                pltpu.sync_copy(bufs[slot], acc.at[idx_v.at[j]], add=True)

            start(0, 0)
            if cpw > 1:
                start(1, 1)

            def loop_body(i, carry):
                j0 = i * 2
                for b in range(2):
                    @pl.when(j0 + b < cpw)
                    def _(b=b):
                        scat(j0 + b, b)

                        @pl.when(j0 + b + 2 < cpw)
                        def _():
                            start(j0 + b + 2, b)
                return carry

            lax.fori_loop(0, (cpw + 1) // 2, loop_body, 0)
            plsc.subcore_barrier()
            pltpu.sync_copy(
                acc.at[pl.ds(tid * nr, nr)],
                out_hbm.at[cid, pl.ds(tid * nr, nr), pl.ds(col, dsh)])
            plsc.subcore_barrier()

    return segsum_kernel, dsh


def _sc_segsum(Y, dst, n):
    m, Dp = Y.shape
    blk = _NW * _CH
    m_pad = ((m + blk - 1) // blk) * blk
    if m_pad != m:
        Y = jnp.concatenate([Y, jnp.zeros((m_pad - m, Dp), Y.dtype)])
        dst = jnp.concatenate([dst, jnp.zeros((m_pad - m,), dst.dtype)])
    call, dsh = _sc_segsum_call(n, Dp, m_pad)
    idx2 = dst.reshape(m_pad // _CH, _CH)
    zero = jnp.zeros((n, dsh), jnp.float32)
    out = call(Y, idx2, zero)
    return out[0] + out[1]


def _sc_gather(table, idx):
    n, D = table.shape
    m = idx.shape[0]
    blk = _NW * _CH
    m_pad = ((m + blk - 1) // blk) * blk
    if m_pad != m:
        idx = jnp.concatenate([idx, jnp.zeros((m_pad - m,), idx.dtype)])
    out = _sc_gather_call(n, D, m_pad)(table, idx)
    return out[:m]


def _bn(x, g, b):
    return x * g / np.sqrt(1.0 + 1e-5) + b


def _tconv(x, edge_index, edge_attr, p):
    src, dst = edge_index[0], edge_index[1]
    n = x.shape[0]
    d = p['Wq'].shape[1]
    q = x @ p['Wq'] + p['bq']
    k = x @ p['Wk'] + p['bk']
    v = x @ p['Wv'] + p['bv']
    e = edge_attr @ p['We'] + p['be']
    kv = jnp.concatenate([k, v], axis=1)
    g = _sc_gather(kv, src)
    qg = _sc_gather(q, dst)
    kj = g[:, :d] + e
    vj = g[:, d:2 * d] + e
    alpha = jnp.sum(qg * kj, axis=-1) / np.sqrt(d)
    amin = jnp.min(alpha)
    rng = jnp.maximum(jnp.max(alpha) - amin, 1e-6)
    band = jnp.minimum(jnp.floor((alpha - amin) * (12.0 / rng)), 11.0)
    z = jnp.exp2(band * 20.0 - 120.0)
    zfull = jnp.concatenate(
        [z[:, None], jnp.zeros((z.shape[0], 15), jnp.float32)], axis=1)
    zseg = _sc_segsum(zfull, dst, n)[:, 0]
    bmax = jnp.floor((jnp.floor(jnp.log2(zseg + 1e-38)) + 120.0) / 20.0)
    c = amin + bmax * (rng / 12.0)
    c_g = _sc_gather(jnp.tile(c[:, None], (1, 16)), dst)[:, 0]
    ex = jnp.exp(alpha - c_g)
    Y = jnp.concatenate(
        [vj * ex[:, None], ex[:, None],
         jnp.zeros((ex.shape[0], 15), jnp.float32)], axis=1)
    S = _sc_segsum(Y, dst, n)
    out = S[:, :d] / (S[:, d:d + 1] + 1e-16)
    return out + x @ p['Ws'] + p['bs']


def _conv1d(t, W, b):
    pad = (W.shape[2] - 1) // 2
    y = jax.lax.conv_general_dilated(t, W, (1,), [(pad, pad)], dimension_numbers=('NCH', 'OIH', 'NCH'))
    return y + b[None, :, None]


def _final_mm_kernel(o_ref, w_ref, b_ref, out_ref):
    out_ref[...] = jnp.dot(o_ref[...], w_ref[...],
                           preferred_element_type=jnp.float32) + b_ref[...]


def _final_mm(o, Wout, bout):
    return pl.pallas_call(
        _final_mm_kernel,
        out_shape=jax.ShapeDtypeStruct((o.shape[0], Wout.shape[1]), jnp.float32),
    )(o, Wout, bout[None, :])


def kernel(x, edge_attr, bag_x, bag_edge_attr, tg_x, tg_edge_attr, t_1D, d_2D, Wf, bf, Wa, ba, Wl, bl, lg, lb, Wd, bd, dg, db, abg1, abg2, abg3, bag1, bag2, bag3, tg1, tg2, tg3, abg_fc1_W, abg_fc1_b, abg_g1, abg_b1, abg_fc2_W, abg_fc2_b, abg_g2, abg_b2, tg_fc1_W, tg_fc1_b, tg_g1, tg_b1, tg_fc2_W, tg_fc2_b, tg_g2, tg_b2, Wout, bout, edge_index, batch_ids, bag_edge_index, tg_edge_index, tg_batch, mask):
    relu = jax.nn.relu
    t_o = _conv1d(t_1D, Wf, bf)
    attention = _conv1d(t_1D, Wa, ba)
    attention = jnp.where(mask[:, None, :], attention, -1e9)
    t_o1 = jnp.sum(t_o * jax.nn.softmax(attention, axis=-1), axis=-1)
    t_o2 = jnp.max(t_o, axis=-1)
    t_o = jnp.concatenate([t_o1, t_o2], axis=-1)
    t_o = _bn(relu(t_o @ Wl + bl), lg, lb)
    d_o = _bn(relu(d_2D @ Wd + bd), dg, db)
    atom_h = relu(_tconv(x, edge_index, edge_attr, abg1))
    edge_h = relu(_tconv(bag_x, bag_edge_index, bag_edge_attr, bag1))
    atom_h = relu(_tconv(atom_h, edge_index, edge_h, abg2))
    edge_h = relu(_tconv(edge_h, bag_edge_index, bag_edge_attr, bag2))
    atom_h = relu(_tconv(atom_h, edge_index, edge_h, abg3))
    edge_h = relu(_tconv(edge_h, bag_edge_index, bag_edge_attr, bag3))
    ah = jax.ops.segment_max(atom_h, batch_ids, num_segments=B)
    ah = jnp.where(jnp.isfinite(ah), ah, 0.0)
    ah = _bn(ah @ abg_fc1_W + abg_fc1_b, abg_g1, abg_b1)
    ah = _bn(ah @ abg_fc2_W + abg_fc2_b, abg_g2, abg_b2)
    AA = relu(_tconv(tg_x, tg_edge_index, tg_edge_attr, tg1))
    AA = relu(_tconv(AA, tg_edge_index, tg_edge_attr, tg2))
    AA = relu(_tconv(AA, tg_edge_index, tg_edge_attr, tg3))
    ssum = jax.ops.segment_sum(AA, tg_batch, num_segments=B)
    cnt = jax.ops.segment_sum(jnp.ones((AA.shape[0],), jnp.float32), tg_batch, num_segments=B)
    AA = ssum / jnp.maximum(cnt, 1.0)[:, None]
    AA = _bn(AA @ tg_fc1_W + tg_fc1_b, tg_g1, tg_b1)
    AA = _bn(AA @ tg_fc2_W + tg_fc2_b, tg_g2, tg_b2)
    o = jnp.concatenate([t_o, d_o, ah, AA], axis=-1)
    return _final_mm(o, Wout, bout)


# speedup vs baseline: 1.0454x; 1.0454x over previous
import functools

import jax
import jax.numpy as jnp
import numpy as np
from jax import lax
from jax.experimental import pallas as pl
from jax.experimental.pallas import tpu as pltpu
from jax.experimental.pallas import tpu_sc as plsc

EMB = 128; B = 64; L = 256; OUT = 2
NN = 10000

_NW = 32
_CH = 128


@functools.lru_cache(maxsize=None)
def _sc_gather_call(n, D, m_pad):
    cpw = m_pad // (_NW * _CH)
    nbuf = 2 if D > 128 else 4
    mesh = plsc.VectorSubcoreMesh(core_axis_name="c", subcore_axis_name="s")

    @functools.partial(
        pl.kernel, mesh=mesh,
        out_type=jax.ShapeDtypeStruct((m_pad, D), jnp.float32),
        compiler_params=pltpu.CompilerParams(use_tc_tiling_on_sc=False),
        scratch_types=[
            pltpu.VMEM((cpw * _CH,), jnp.int32),
        ] + [pltpu.VMEM((_CH, D), jnp.float32)] * nbuf
          + [pltpu.SemaphoreType.DMA] * nbuf,
    )
    def gather_kernel(table_hbm, idx_hbm, out_hbm, idx_v, *bufsems):
        bufs = bufsems[:nbuf]
        sems = bufsems[nbuf:]
        wid = lax.axis_index("s") * 2 + lax.axis_index("c")
        base = wid * (cpw * _CH)
        pltpu.sync_copy(idx_hbm.at[pl.ds(base, cpw * _CH)], idx_v)

        def start(j, slot):
            pltpu.async_copy(
                table_hbm.at[idx_v.at[pl.ds(j * _CH, _CH)]], bufs[slot], sems[slot])

        def drain(j, slot):
            pltpu.make_async_copy(
                table_hbm.at[idx_v.at[pl.ds(j * _CH, _CH)]], bufs[slot],
                sems[slot]).wait()
            pltpu.sync_copy(bufs[slot], out_hbm.at[pl.ds(base + j * _CH, _CH)])

        for b in range(nbuf):
            if b < cpw:
                start(b, b)

        def loop_body(i, carry):
            j0 = i * nbuf
            for b in range(nbuf):
                @pl.when(j0 + b < cpw)
                def _(b=b):
                    drain(j0 + b, b)

                    @pl.when(j0 + b + nbuf < cpw)
                    def _():
                        start(j0 + b + nbuf, b)
            return carry

        lax.fori_loop(0, (cpw + nbuf - 1) // nbuf, loop_body, 0)

    return gather_kernel


def _pick_dsh(n, Dp, cpw):
    for dsh in sorted({d for d in range(8, Dp + 1, 8) if Dp % d == 0},
                      reverse=True):
        words = n * dsh + 16 * (cpw * _CH + 8 * _CH * dsh)
        if words <= 1_950_000:
            return dsh
    raise ValueError("no feasible shard width")


@functools.lru_cache(maxsize=None)
def _sc_segsum_call(n, Dp, m_pad):
    cpw = m_pad // (_NW * _CH)
    dsh = _pick_dsh(n, Dp, cpw)
    nshard = Dp // dsh
    nr = n // 16
    mesh = plsc.VectorSubcoreMesh(core_axis_name="c", subcore_axis_name="s")

    @functools.partial(
        pl.kernel, mesh=mesh,
        out_type=jax.ShapeDtypeStruct((2, n, Dp), jnp.float32),
        compiler_params=pltpu.CompilerParams(use_tc_tiling_on_sc=False),
        scratch_types=[
            pltpu.VMEM((cpw, _CH), jnp.int32),
            pltpu.VMEM((4 * _CH, dsh), jnp.float32),
            pltpu.VMEM((4 * _CH, dsh), jnp.float32),
            pltpu.VMEM_SHARED((n, dsh), jnp.float32),
            pltpu.SemaphoreType.DMA,
            pltpu.SemaphoreType.DMA,
            pltpu.SemaphoreType.DMA,
        ],
    )
    def segsum_kernel(y_hbm, idx_hbm, zero_hbm, out_hbm,
                      idx_v, buf0, buf1, acc, sem0, sem1, sems):
        cid = lax.axis_index("c")
        tid = lax.axis_index("s")
        wid = tid * 2 + cid
        chunk0 = wid * cpw
        ngen, tail = cpw // 4, cpw % 4
        pltpu.sync_copy(idx_hbm.at[pl.ds(chunk0, cpw)], idx_v)
        bufs = (buf0, buf1)
        gsems = (sem0, sem1)

        for s in range(nshard):
            col = s * dsh
            pltpu.sync_copy(zero_hbm.at[pl.ds(tid * nr, nr)],
                            acc.at[pl.ds(tid * nr, nr)])
            plsc.subcore_barrier()

            def start(g, slot, nch=4):
                pltpu.async_copy(
                    y_hbm.at[pl.ds((chunk0 + g * 4) * _CH, nch * _CH),
                             pl.ds(col, dsh)],
                    bufs[slot].at[pl.ds(0, nch * _CH)], gsems[slot])

            def process(g, slot, nch=4):
                pltpu.make_async_copy(
                    y_hbm.at[pl.ds((chunk0 + g * 4) * _CH, nch * _CH),
                             pl.ds(col, dsh)],
                    bufs[slot].at[pl.ds(0, nch * _CH)], gsems[slot]).wait()
                for b in range(nch):
                    pltpu.async_copy(bufs[slot].at[pl.ds(b * _CH, _CH)],
                                     acc.at[idx_v.at[g * 4 + b]], sems,
                                     add=True)
                for b in range(nch):
                    pltpu.make_async_copy(
                        bufs[slot].at[pl.ds(b * _CH, _CH)],
                        acc.at[idx_v.at[g * 4 + b]], sems).wait()

            if ngen > 0:
                start(0, 0)
            if ngen > 1:
                start(1, 1)

            def loop_body(i, carry):
                g0 = i * 2
                for b in range(2):
                    @pl.when(g0 + b < ngen)
                    def _(b=b):
                        process(g0 + b, b)

                        @pl.when(g0 + b + 2 < ngen)
                        def _():
                            start(g0 + b + 2, b)
                return carry

            lax.fori_loop(0, (ngen + 1) // 2, loop_body, 0)
            if tail:
                start(ngen, 0, tail)
                process(ngen, 0, tail)
            plsc.subcore_barrier()
            pltpu.sync_copy(
                acc.at[pl.ds(tid * nr, nr)],
                out_hbm.at[cid, pl.ds(tid * nr, nr), pl.ds(col, dsh)])
            plsc.subcore_barrier()

    return segsum_kernel, dsh


def _sc_segsum(Y, dst, n):
    m, Dp = Y.shape
    blk = _NW * _CH
    m_pad = ((m + blk - 1) // blk) * blk
    if m_pad != m:
        Y = jnp.concatenate([Y, jnp.zeros((m_pad - m, Dp), Y.dtype)])
        dst = jnp.concatenate([dst, jnp.zeros((m_pad - m,), dst.dtype)])
    call, dsh = _sc_segsum_call(n, Dp, m_pad)
    idx2 = dst.reshape(m_pad // _CH, _CH)
    zero = jnp.zeros((n, dsh), jnp.float32)
    out = call(Y, idx2, zero)
    return out[0] + out[1]


def _sc_gather(table, idx):
    n, D = table.shape
    m = idx.shape[0]
    blk = _NW * _CH
    m_pad = ((m + blk - 1) // blk) * blk
    if m_pad != m:
        idx = jnp.concatenate([idx, jnp.zeros((m_pad - m,), idx.dtype)])
    out = _sc_gather_call(n, D, m_pad)(table, idx)
    return out[:m]


def _bn(x, g, b):
    return x * g / np.sqrt(1.0 + 1e-5) + b


def _tconv(x, edge_index, edge_attr, p):
    src, dst = edge_index[0], edge_index[1]
    n = x.shape[0]
    d = p['Wq'].shape[1]
    q = x @ p['Wq'] + p['bq']
    k = x @ p['Wk'] + p['bk']
    v = x @ p['Wv'] + p['bv']
    e = edge_attr @ p['We'] + p['be']
    kv = jnp.concatenate([k, v], axis=1)
    g = _sc_gather(kv, src)
    qg = _sc_gather(q, dst)
    kj = g[:, :d] + e
    vj = g[:, d:2 * d] + e
    alpha = jnp.sum(qg * kj, axis=-1) / np.sqrt(d)
    amax = jax.ops.segment_max(alpha, dst, num_segments=n)
    amax = jnp.where(jnp.isfinite(amax), amax, 0.0)
    amax_g = _sc_gather(jnp.tile(amax[:, None], (1, 16)), dst)[:, 0]
    ex = jnp.exp(alpha - amax_g)
    Y = jnp.concatenate(
        [vj * ex[:, None], ex[:, None],
         jnp.zeros((ex.shape[0], 15), jnp.float32)], axis=1)
    S = _sc_segsum(Y, dst, n)
    out = S[:, :d] / (S[:, d:d + 1] + 1e-16)
    return out + x @ p['Ws'] + p['bs']


def _conv1d(t, W, b):
    pad = (W.shape[2] - 1) // 2
    y = jax.lax.conv_general_dilated(t, W, (1,), [(pad, pad)], dimension_numbers=('NCH', 'OIH', 'NCH'))
    return y + b[None, :, None]


def _final_mm_kernel(o_ref, w_ref, b_ref, out_ref):
    out_ref[...] = jnp.dot(o_ref[...], w_ref[...],
                           preferred_element_type=jnp.float32) + b_ref[...]


def _final_mm(o, Wout, bout):
    return pl.pallas_call(
        _final_mm_kernel,
        out_shape=jax.ShapeDtypeStruct((o.shape[0], Wout.shape[1]), jnp.float32),
    )(o, Wout, bout[None, :])


def kernel(x, edge_attr, bag_x, bag_edge_attr, tg_x, tg_edge_attr, t_1D, d_2D, Wf, bf, Wa, ba, Wl, bl, lg, lb, Wd, bd, dg, db, abg1, abg2, abg3, bag1, bag2, bag3, tg1, tg2, tg3, abg_fc1_W, abg_fc1_b, abg_g1, abg_b1, abg_fc2_W, abg_fc2_b, abg_g2, abg_b2, tg_fc1_W, tg_fc1_b, tg_g1, tg_b1, tg_fc2_W, tg_fc2_b, tg_g2, tg_b2, Wout, bout, edge_index, batch_ids, bag_edge_index, tg_edge_index, tg_batch, mask):
    relu = jax.nn.relu
    t_o = _conv1d(t_1D, Wf, bf)
    attention = _conv1d(t_1D, Wa, ba)
    attention = jnp.where(mask[:, None, :], attention, -1e9)
    t_o1 = jnp.sum(t_o * jax.nn.softmax(attention, axis=-1), axis=-1)
    t_o2 = jnp.max(t_o, axis=-1)
    t_o = jnp.concatenate([t_o1, t_o2], axis=-1)
    t_o = _bn(relu(t_o @ Wl + bl), lg, lb)
    d_o = _bn(relu(d_2D @ Wd + bd), dg, db)
    atom_h = relu(_tconv(x, edge_index, edge_attr, abg1))
    edge_h = relu(_tconv(bag_x, bag_edge_index, bag_edge_attr, bag1))
    atom_h = relu(_tconv(atom_h, edge_index, edge_h, abg2))
    edge_h = relu(_tconv(edge_h, bag_edge_index, bag_edge_attr, bag2))
    atom_h = relu(_tconv(atom_h, edge_index, edge_h, abg3))
    edge_h = relu(_tconv(edge_h, bag_edge_index, bag_edge_attr, bag3))
    ah = jax.ops.segment_max(atom_h, batch_ids, num_segments=B)
    ah = jnp.where(jnp.isfinite(ah), ah, 0.0)
    ah = _bn(ah @ abg_fc1_W + abg_fc1_b, abg_g1, abg_b1)
    ah = _bn(ah @ abg_fc2_W + abg_fc2_b, abg_g2, abg_b2)
    AA = relu(_tconv(tg_x, tg_edge_index, tg_edge_attr, tg1))
    AA = relu(_tconv(AA, tg_edge_index, tg_edge_attr, tg2))
    AA = relu(_tconv(AA, tg_edge_index, tg_edge_attr, tg3))
    ssum = jax.ops.segment_sum(AA, tg_batch, num_segments=B)
    cnt = jax.ops.segment_sum(jnp.ones((AA.shape[0],), jnp.float32), tg_batch, num_segments=B)
    AA = ssum / jnp.maximum(cnt, 1.0)[:, None]
    AA = _bn(AA @ tg_fc1_W + tg_fc1_b, tg_g1, tg_b1)
    AA = _bn(AA @ tg_fc2_W + tg_fc2_b, tg_g2, tg_b2)
    o = jnp.concatenate([t_o, d_o, ah, AA], axis=-1)
    return _final_mm(o, Wout, bout)


# speedup vs baseline: 1.0745x; 1.0278x over previous
import functools

import jax
import jax.numpy as jnp
import numpy as np
from jax import lax
from jax.experimental import pallas as pl
from jax.experimental.pallas import tpu as pltpu
from jax.experimental.pallas import tpu_sc as plsc

EMB = 128; B = 64; L = 256; OUT = 2
NN = 10000

_NW = 32
_CH = 128


@functools.lru_cache(maxsize=None)
def _sc_gather_call(n, D, m_pad):
    cpw = m_pad // (_NW * _CH)
    nbuf = 2 if D > 128 else 4
    mesh = plsc.VectorSubcoreMesh(core_axis_name="c", subcore_axis_name="s")

    @functools.partial(
        pl.kernel, mesh=mesh,
        out_type=jax.ShapeDtypeStruct((m_pad, D), jnp.float32),
        compiler_params=pltpu.CompilerParams(use_tc_tiling_on_sc=False),
        scratch_types=[
            pltpu.VMEM((cpw * _CH,), jnp.int32),
        ] + [pltpu.VMEM((_CH, D), jnp.float32)] * nbuf
          + [pltpu.SemaphoreType.DMA] * nbuf,
    )
    def gather_kernel(table_hbm, idx_hbm, out_hbm, idx_v, *bufsems):
        bufs = bufsems[:nbuf]
        sems = bufsems[nbuf:]
        wid = lax.axis_index("s") * 2 + lax.axis_index("c")
        base = wid * (cpw * _CH)
        pltpu.sync_copy(idx_hbm.at[pl.ds(base, cpw * _CH)], idx_v)

        def start(j, slot):
            pltpu.async_copy(
                table_hbm.at[idx_v.at[pl.ds(j * _CH, _CH)]], bufs[slot], sems[slot])

        def drain(j, slot):
            pltpu.make_async_copy(
                table_hbm.at[idx_v.at[pl.ds(j * _CH, _CH)]], bufs[slot],
                sems[slot]).wait()
            pltpu.sync_copy(bufs[slot], out_hbm.at[pl.ds(base + j * _CH, _CH)])

        for b in range(nbuf):
            if b < cpw:
                start(b, b)

        def loop_body(i, carry):
            j0 = i * nbuf
            for b in range(nbuf):
                @pl.when(j0 + b < cpw)
                def _(b=b):
                    drain(j0 + b, b)

                    @pl.when(j0 + b + nbuf < cpw)
                    def _():
                        start(j0 + b + nbuf, b)
            return carry

        lax.fori_loop(0, (cpw + nbuf - 1) // nbuf, loop_body, 0)

    return gather_kernel


def _pick_dsh(n, Dp, cpw):
    for dsh in sorted({d for d in range(8, Dp + 1, 8) if Dp % d == 0},
                      reverse=True):
        words = n * dsh + 16 * (cpw * _CH + 8 * _CH * dsh)
        if words <= 1_950_000:
            return dsh
    raise ValueError("no feasible shard width")


@functools.lru_cache(maxsize=None)
def _sc_segsum_call(n, Dp, m_pad):
    cpw = m_pad // (_NW * _CH)
    dsh = _pick_dsh(n, Dp, cpw)
    nshard = Dp // dsh
    nr = n // 16
    mesh = plsc.VectorSubcoreMesh(core_axis_name="c", subcore_axis_name="s")

    @functools.partial(
        pl.kernel, mesh=mesh,
        out_type=jax.ShapeDtypeStruct((2, n, Dp), jnp.float32),
        compiler_params=pltpu.CompilerParams(use_tc_tiling_on_sc=False),
        scratch_types=[
            pltpu.VMEM((cpw, _CH), jnp.int32),
            pltpu.VMEM((4 * _CH, dsh), jnp.float32),
            pltpu.VMEM((4 * _CH, dsh), jnp.float32),
            pltpu.VMEM_SHARED((n, dsh), jnp.float32),
            pltpu.SemaphoreType.DMA,
            pltpu.SemaphoreType.DMA,
            pltpu.SemaphoreType.DMA,
        ],
    )
    def segsum_kernel(y_hbm, idx_hbm, zero_hbm, out_hbm,
                      idx_v, buf0, buf1, acc, sem0, sem1, sems):
        cid = lax.axis_index("c")
        tid = lax.axis_index("s")
        wid = tid * 2 + cid
        chunk0 = wid * cpw
        ngen, tail = cpw // 4, cpw % 4
        pltpu.sync_copy(idx_hbm.at[pl.ds(chunk0, cpw)], idx_v)
        bufs = (buf0, buf1)
        gsems = (sem0, sem1)

        for s in range(nshard):
            col = s * dsh
            pltpu.sync_copy(zero_hbm.at[pl.ds(tid * nr, nr)],
                            acc.at[pl.ds(tid * nr, nr)])
            plsc.subcore_barrier()

            def start(g, slot, nch=4):
                pltpu.async_copy(
                    y_hbm.at[pl.ds((chunk0 + g * 4) * _CH, nch * _CH),
                             pl.ds(col, dsh)],
                    bufs[slot].at[pl.ds(0, nch * _CH)], gsems[slot])

            def process(g, slot, nch=4):
                pltpu.make_async_copy(
                    y_hbm.at[pl.ds((chunk0 + g * 4) * _CH, nch * _CH),
                             pl.ds(col, dsh)],
                    bufs[slot].at[pl.ds(0, nch * _CH)], gsems[slot]).wait()
                for b in range(nch):
                    pltpu.async_copy(bufs[slot].at[pl.ds(b * _CH, _CH)],
                                     acc.at[idx_v.at[g * 4 + b]], sems,
                                     add=True)
                for b in range(nch):
                    pltpu.make_async_copy(
                        bufs[slot].at[pl.ds(b * _CH, _CH)],
                        acc.at[idx_v.at[g * 4 + b]], sems).wait()

            if ngen > 0:
                start(0, 0)
            if ngen > 1:
                start(1, 1)

            def loop_body(i, carry):
                g0 = i * 2
                for b in range(2):
                    @pl.when(g0 + b < ngen)
                    def _(b=b):
                        process(g0 + b, b)

                        @pl.when(g0 + b + 2 < ngen)
                        def _():
                            start(g0 + b + 2, b)
                return carry

            lax.fori_loop(0, (ngen + 1) // 2, loop_body, 0)
            if tail:
                start(ngen, 0, tail)
                process(ngen, 0, tail)
            plsc.subcore_barrier()
            pltpu.sync_copy(
                acc.at[pl.ds(tid * nr, nr)],
                out_hbm.at[cid, pl.ds(tid * nr, nr), pl.ds(col, dsh)])
            plsc.subcore_barrier()

    return segsum_kernel, dsh


def _sc_segsum(Y, dst, n):
    m, Dp = Y.shape
    blk = _NW * _CH
    m_pad = ((m + blk - 1) // blk) * blk
    if m_pad != m:
        Y = jnp.concatenate([Y, jnp.zeros((m_pad - m, Dp), Y.dtype)])
        dst = jnp.concatenate([dst, jnp.zeros((m_pad - m,), dst.dtype)])
    call, dsh = _sc_segsum_call(n, Dp, m_pad)
    idx2 = dst.reshape(m_pad // _CH, _CH)
    zero = jnp.zeros((n, dsh), jnp.float32)
    out = call(Y, idx2, zero)
    return out[0] + out[1]


def _sc_gather(table, idx):
    n, D = table.shape
    m = idx.shape[0]
    blk = _NW * _CH
    m_pad = ((m + blk - 1) // blk) * blk
    if m_pad != m:
        idx = jnp.concatenate([idx, jnp.zeros((m_pad - m,), idx.dtype)])
    out = _sc_gather_call(n, D, m_pad)(table, idx)
    return out[:m]


def _bn(x, g, b):
    return x * g / np.sqrt(1.0 + 1e-5) + b


def _tconv(x, src, dst, e, pad, p):
    n = x.shape[0]
    d = p['Wq'].shape[1]
    q = x @ p['Wq'] + p['bq']
    k = x @ p['Wk'] + p['bk']
    v = x @ p['Wv'] + p['bv']
    kv = jnp.concatenate([k, v], axis=1)
    g = _sc_gather(kv, src)
    qg = _sc_gather(q, dst)
    kj = g[:, :d] + e
    vj = g[:, d:2 * d] + e
    alpha = jnp.sum(qg * kj, axis=-1) / np.sqrt(d)
    alpha = jnp.where(pad, -jnp.inf, alpha)
    amax = jax.ops.segment_max(alpha, dst, num_segments=n)
    amax = jnp.where(jnp.isfinite(amax), amax, 0.0)
    amax_g = _sc_gather(jnp.tile(amax[:, None], (1, 16)), dst)[:, 0]
    ex = jnp.exp(alpha - amax_g)
    Y = jnp.concatenate(
        [vj * ex[:, None], ex[:, None],
         jnp.zeros((ex.shape[0], 15), jnp.float32)], axis=1)
    S = _sc_segsum(Y, dst, n)
    out = S[:, :d] / (S[:, d:d + 1] + 1e-16)
    return out + x @ p['Ws'] + p['bs']


def _pad_graph(edge_index, m):
    blk = _NW * _CH
    m_pad = ((m + blk - 1) // blk) * blk
    src = jnp.pad(edge_index[0], (0, m_pad - m))
    dst = jnp.pad(edge_index[1], (0, m_pad - m))
    pad = jnp.arange(m_pad) >= m
    return src, dst, pad, m_pad


def _conv1d(t, W, b):
    pad = (W.shape[2] - 1) // 2
    y = jax.lax.conv_general_dilated(t, W, (1,), [(pad, pad)], dimension_numbers=('NCH', 'OIH', 'NCH'))
    return y + b[None, :, None]


def _final_mm_kernel(o_ref, w_ref, b_ref, out_ref):
    out_ref[...] = jnp.dot(o_ref[...], w_ref[...],
                           preferred_element_type=jnp.float32) + b_ref[...]


def _final_mm(o, Wout, bout):
    return pl.pallas_call(
        _final_mm_kernel,
        out_shape=jax.ShapeDtypeStruct((o.shape[0], Wout.shape[1]), jnp.float32),
    )(o, Wout, bout[None, :])


def kernel(x, edge_attr, bag_x, bag_edge_attr, tg_x, tg_edge_attr, t_1D, d_2D, Wf, bf, Wa, ba, Wl, bl, lg, lb, Wd, bd, dg, db, abg1, abg2, abg3, bag1, bag2, bag3, tg1, tg2, tg3, abg_fc1_W, abg_fc1_b, abg_g1, abg_b1, abg_fc2_W, abg_fc2_b, abg_g2, abg_b2, tg_fc1_W, tg_fc1_b, tg_g1, tg_b1, tg_fc2_W, tg_fc2_b, tg_g2, tg_b2, Wout, bout, edge_index, batch_ids, bag_edge_index, tg_edge_index, tg_batch, mask):
    relu = jax.nn.relu
    t_o = _conv1d(t_1D, Wf, bf)
    attention = _conv1d(t_1D, Wa, ba)
    attention = jnp.where(mask[:, None, :], attention, -1e9)
    t_o1 = jnp.sum(t_o * jax.nn.softmax(attention, axis=-1), axis=-1)
    t_o2 = jnp.max(t_o, axis=-1)
    t_o = jnp.concatenate([t_o1, t_o2], axis=-1)
    t_o = _bn(relu(t_o @ Wl + bl), lg, lb)
    d_o = _bn(relu(d_2D @ Wd + bd), dg, db)
    def eproj(ea, m_pad, p):
        return jnp.pad(ea, ((0, m_pad - ea.shape[0]), (0, 0))) @ p['We'] + p['be']

    a_src, a_dst, a_pad, a_mp = _pad_graph(edge_index, edge_index.shape[1])
    b_src, b_dst, b_pad, b_mp = _pad_graph(bag_edge_index, bag_edge_index.shape[1])
    t_src, t_dst, t_pad, t_mp = _pad_graph(tg_edge_index, tg_edge_index.shape[1])

    atom_h = relu(_tconv(x, a_src, a_dst, eproj(edge_attr, a_mp, abg1), a_pad, abg1))
    edge_h = relu(_tconv(bag_x, b_src, b_dst, eproj(bag_edge_attr, b_mp, bag1), b_pad, bag1))
    atom_h = relu(_tconv(atom_h, a_src, a_dst, eproj(edge_h, a_mp, abg2), a_pad, abg2))
    edge_h = relu(_tconv(edge_h, b_src, b_dst, eproj(bag_edge_attr, b_mp, bag2), b_pad, bag2))
    atom_h = relu(_tconv(atom_h, a_src, a_dst, eproj(edge_h, a_mp, abg3), a_pad, abg3))
    edge_h = relu(_tconv(edge_h, b_src, b_dst, eproj(bag_edge_attr, b_mp, bag3), b_pad, bag3))
    ah = jax.ops.segment_max(atom_h, batch_ids, num_segments=B)
    ah = jnp.where(jnp.isfinite(ah), ah, 0.0)
    ah = _bn(ah @ abg_fc1_W + abg_fc1_b, abg_g1, abg_b1)
    ah = _bn(ah @ abg_fc2_W + abg_fc2_b, abg_g2, abg_b2)
    AA = relu(_tconv(tg_x, t_src, t_dst, eproj(tg_edge_attr, t_mp, tg1), t_pad, tg1))
    AA = relu(_tconv(AA, t_src, t_dst, eproj(tg_edge_attr, t_mp, tg2), t_pad, tg2))
    AA = relu(_tconv(AA, t_src, t_dst, eproj(tg_edge_attr, t_mp, tg3), t_pad, tg3))
    ssum = jax.ops.segment_sum(AA, tg_batch, num_segments=B)
    cnt = jax.ops.segment_sum(jnp.ones((AA.shape[0],), jnp.float32), tg_batch, num_segments=B)
    AA = ssum / jnp.maximum(cnt, 1.0)[:, None]
    AA = _bn(AA @ tg_fc1_W + tg_fc1_b, tg_g1, tg_b1)
    AA = _bn(AA @ tg_fc2_W + tg_fc2_b, tg_g2, tg_b2)
    o = jnp.concatenate([t_o, d_o, ah, AA], axis=-1)
    return _final_mm(o, Wout, bout)


# speedup vs baseline: 1.0766x; 1.0020x over previous
import functools

import jax
import jax.numpy as jnp
import numpy as np
from jax import lax
from jax.experimental import pallas as pl
from jax.experimental.pallas import tpu as pltpu
from jax.experimental.pallas import tpu_sc as plsc

EMB = 128; B = 64; L = 256; OUT = 2
NN = 10000

_NW = 32
_CH = 128


@functools.lru_cache(maxsize=None)
def _sc_gather_call(n, D, m_pad):
    cpw = m_pad // (_NW * _CH)
    nch = max(1, min(4, 60000 // (_CH * D)))
    ngen, tail = cpw // nch, cpw % nch
    mesh = plsc.VectorSubcoreMesh(core_axis_name="c", subcore_axis_name="s")

    @functools.partial(
        pl.kernel, mesh=mesh,
        out_type=jax.ShapeDtypeStruct((m_pad, D), jnp.float32),
        compiler_params=pltpu.CompilerParams(use_tc_tiling_on_sc=False),
        scratch_types=[
            pltpu.VMEM((cpw * _CH,), jnp.int32),
            pltpu.VMEM((nch * _CH, D), jnp.float32),
            pltpu.VMEM((nch * _CH, D), jnp.float32),
            pltpu.SemaphoreType.DMA,
            pltpu.SemaphoreType.DMA,
        ],
    )
    def gather_kernel(table_hbm, idx_hbm, out_hbm, idx_v, buf0, buf1,
                      sem0, sem1):
        bufs = (buf0, buf1)
        sems = (sem0, sem1)
        wid = lax.axis_index("s") * 2 + lax.axis_index("c")
        base = wid * (cpw * _CH)
        pltpu.sync_copy(idx_hbm.at[pl.ds(base, cpw * _CH)], idx_v)

        def start(g, slot, nc=nch):
            for b in range(nc):
                pltpu.async_copy(
                    table_hbm.at[idx_v.at[pl.ds((g * nch + b) * _CH, _CH)]],
                    bufs[slot].at[pl.ds(b * _CH, _CH)], sems[slot])

        def drain(g, slot, nc=nch):
            for b in range(nc):
                pltpu.make_async_copy(
                    table_hbm.at[idx_v.at[pl.ds((g * nch + b) * _CH, _CH)]],
                    bufs[slot].at[pl.ds(b * _CH, _CH)], sems[slot]).wait()
            pltpu.sync_copy(
                bufs[slot].at[pl.ds(0, nc * _CH)],
                out_hbm.at[pl.ds(base + g * nch * _CH, nc * _CH)])

        if ngen > 0:
            start(0, 0)
        if ngen > 1:
            start(1, 1)

        def loop_body(i, carry):
            g0 = i * 2
            for b in range(2):
                @pl.when(g0 + b < ngen)
                def _(b=b):
                    drain(g0 + b, b)

                    @pl.when(g0 + b + 2 < ngen)
                    def _():
                        start(g0 + b + 2, b)
            return carry

        lax.fori_loop(0, (ngen + 1) // 2, loop_body, 0)
        if tail:
            start(ngen, 0, tail)
            drain(ngen, 0, tail)

    return gather_kernel


def _pick_dsh(n, Dp, cpw):
    for dsh in sorted({d for d in range(8, Dp + 1, 8) if Dp % d == 0},
                      reverse=True):
        words = n * dsh + 16 * (cpw * _CH + 8 * _CH * dsh)
        if words <= 1_950_000:
            return dsh
    raise ValueError("no feasible shard width")


@functools.lru_cache(maxsize=None)
def _sc_segsum_call(n, Dp, m_pad):
    cpw = m_pad // (_NW * _CH)
    dsh = _pick_dsh(n, Dp, cpw)
    nshard = Dp // dsh
    nr = n // 16
    mesh = plsc.VectorSubcoreMesh(core_axis_name="c", subcore_axis_name="s")

    @functools.partial(
        pl.kernel, mesh=mesh,
        out_type=jax.ShapeDtypeStruct((2, n, Dp), jnp.float32),
        compiler_params=pltpu.CompilerParams(use_tc_tiling_on_sc=False),
        scratch_types=[
            pltpu.VMEM((cpw, _CH), jnp.int32),
            pltpu.VMEM((4 * _CH, dsh), jnp.float32),
            pltpu.VMEM((4 * _CH, dsh), jnp.float32),
            pltpu.VMEM_SHARED((n, dsh), jnp.float32),
            pltpu.SemaphoreType.DMA,
            pltpu.SemaphoreType.DMA,
            pltpu.SemaphoreType.DMA,
        ],
    )
    def segsum_kernel(y_hbm, idx_hbm, zero_hbm, out_hbm,
                      idx_v, buf0, buf1, acc, sem0, sem1, sems):
        cid = lax.axis_index("c")
        tid = lax.axis_index("s")
        wid = tid * 2 + cid
        chunk0 = wid * cpw
        ngen, tail = cpw // 4, cpw % 4
        pltpu.sync_copy(idx_hbm.at[pl.ds(chunk0, cpw)], idx_v)
        bufs = (buf0, buf1)
        gsems = (sem0, sem1)

        for s in range(nshard):
            col = s * dsh
            pltpu.sync_copy(zero_hbm.at[pl.ds(tid * nr, nr)],
                            acc.at[pl.ds(tid * nr, nr)])
            plsc.subcore_barrier()

            def start(g, slot, nch=4):
                pltpu.async_copy(
                    y_hbm.at[pl.ds((chunk0 + g * 4) * _CH, nch * _CH),
                             pl.ds(col, dsh)],
                    bufs[slot].at[pl.ds(0, nch * _CH)], gsems[slot])

            def process(g, slot, nch=4):
                pltpu.make_async_copy(
                    y_hbm.at[pl.ds((chunk0 + g * 4) * _CH, nch * _CH),
                             pl.ds(col, dsh)],
                    bufs[slot].at[pl.ds(0, nch * _CH)], gsems[slot]).wait()
                for b in range(nch):
                    pltpu.async_copy(bufs[slot].at[pl.ds(b * _CH, _CH)],
                                     acc.at[idx_v.at[g * 4 + b]], sems,
                                     add=True)
                for b in range(nch):
                    pltpu.make_async_copy(
                        bufs[slot].at[pl.ds(b * _CH, _CH)],
                        acc.at[idx_v.at[g * 4 + b]], sems).wait()

            if ngen > 0:
                start(0, 0)
            if ngen > 1:
                start(1, 1)

            def loop_body(i, carry):
                g0 = i * 2
                for b in range(2):
                    @pl.when(g0 + b < ngen)
                    def _(b=b):
                        process(g0 + b, b)

                        @pl.when(g0 + b + 2 < ngen)
                        def _():
                            start(g0 + b + 2, b)
                return carry

            lax.fori_loop(0, (ngen + 1) // 2, loop_body, 0)
            if tail:
                start(ngen, 0, tail)
                process(ngen, 0, tail)
            plsc.subcore_barrier()
            pltpu.sync_copy(
                acc.at[pl.ds(tid * nr, nr)],
                out_hbm.at[cid, pl.ds(tid * nr, nr), pl.ds(col, dsh)])
            plsc.subcore_barrier()

    return segsum_kernel, dsh


def _sc_segsum(Y, dst, n):
    m, Dp = Y.shape
    blk = _NW * _CH
    m_pad = ((m + blk - 1) // blk) * blk
    if m_pad != m:
        Y = jnp.concatenate([Y, jnp.zeros((m_pad - m, Dp), Y.dtype)])
        dst = jnp.concatenate([dst, jnp.zeros((m_pad - m,), dst.dtype)])
    call, dsh = _sc_segsum_call(n, Dp, m_pad)
    idx2 = dst.reshape(m_pad // _CH, _CH)
    zero = jnp.zeros((n, dsh), jnp.float32)
    out = call(Y, idx2, zero)
    return out[0] + out[1]


def _sc_gather(table, idx):
    n, D = table.shape
    m = idx.shape[0]
    blk = _NW * _CH
    m_pad = ((m + blk - 1) // blk) * blk
    if m_pad != m:
        idx = jnp.concatenate([idx, jnp.zeros((m_pad - m,), idx.dtype)])
    out = _sc_gather_call(n, D, m_pad)(table, idx)
    return out[:m]


def _bn(x, g, b):
    return x * g / np.sqrt(1.0 + 1e-5) + b


def _tconv(x, src, dst, e, pad, p):
    n = x.shape[0]
    d = p['Wq'].shape[1]
    q = x @ p['Wq'] + p['bq']
    k = x @ p['Wk'] + p['bk']
    v = x @ p['Wv'] + p['bv']
    kv = jnp.concatenate([k, v], axis=1)
    g = _sc_gather(kv, src)
    qg = _sc_gather(q, dst)
    kj = g[:, :d] + e
    vj = g[:, d:2 * d] + e
    alpha = jnp.sum(qg * kj, axis=-1) / np.sqrt(d)
    alpha = jnp.where(pad, -jnp.inf, alpha)
    amax = jax.ops.segment_max(alpha, dst, num_segments=n)
    amax = jnp.where(jnp.isfinite(amax), amax, 0.0)
    amax_g = _sc_gather(jnp.tile(amax[:, None], (1, 16)), dst)[:, 0]
    ex = jnp.exp(alpha - amax_g)
    Y = jnp.concatenate(
        [vj * ex[:, None], ex[:, None],
         jnp.zeros((ex.shape[0], 15), jnp.float32)], axis=1)
    S = _sc_segsum(Y, dst, n)
    out = S[:, :d] / (S[:, d:d + 1] + 1e-16)
    return out + x @ p['Ws'] + p['bs']


def _pad_graph(edge_index, m):
    blk = _NW * _CH
    m_pad = ((m + blk - 1) // blk) * blk
    src = jnp.pad(edge_index[0], (0, m_pad - m))
    dst = jnp.pad(edge_index[1], (0, m_pad - m))
    pad = jnp.arange(m_pad) >= m
    return src, dst, pad, m_pad


def _conv1d(t, W, b):
    pad = (W.shape[2] - 1) // 2
    y = jax.lax.conv_general_dilated(t, W, (1,), [(pad, pad)], dimension_numbers=('NCH', 'OIH', 'NCH'))
    return y + b[None, :, None]


def _final_mm_kernel(o_ref, w_ref, b_ref, out_ref):
    out_ref[...] = jnp.dot(o_ref[...], w_ref[...],
                           preferred_element_type=jnp.float32) + b_ref[...]


def _final_mm(o, Wout, bout):
    return pl.pallas_call(
        _final_mm_kernel,
        out_shape=jax.ShapeDtypeStruct((o.shape[0], Wout.shape[1]), jnp.float32),
    )(o, Wout, bout[None, :])


def kernel(x, edge_attr, bag_x, bag_edge_attr, tg_x, tg_edge_attr, t_1D, d_2D, Wf, bf, Wa, ba, Wl, bl, lg, lb, Wd, bd, dg, db, abg1, abg2, abg3, bag1, bag2, bag3, tg1, tg2, tg3, abg_fc1_W, abg_fc1_b, abg_g1, abg_b1, abg_fc2_W, abg_fc2_b, abg_g2, abg_b2, tg_fc1_W, tg_fc1_b, tg_g1, tg_b1, tg_fc2_W, tg_fc2_b, tg_g2, tg_b2, Wout, bout, edge_index, batch_ids, bag_edge_index, tg_edge_index, tg_batch, mask):
    relu = jax.nn.relu
    t_o = _conv1d(t_1D, Wf, bf)
    attention = _conv1d(t_1D, Wa, ba)
    attention = jnp.where(mask[:, None, :], attention, -1e9)
    t_o1 = jnp.sum(t_o * jax.nn.softmax(attention, axis=-1), axis=-1)
    t_o2 = jnp.max(t_o, axis=-1)
    t_o = jnp.concatenate([t_o1, t_o2], axis=-1)
    t_o = _bn(relu(t_o @ Wl + bl), lg, lb)
    d_o = _bn(relu(d_2D @ Wd + bd), dg, db)
    def eproj(ea, m_pad, p):
        return jnp.pad(ea, ((0, m_pad - ea.shape[0]), (0, 0))) @ p['We'] + p['be']

    a_src, a_dst, a_pad, a_mp = _pad_graph(edge_index, edge_index.shape[1])
    b_src, b_dst, b_pad, b_mp = _pad_graph(bag_edge_index, bag_edge_index.shape[1])
    t_src, t_dst, t_pad, t_mp = _pad_graph(tg_edge_index, tg_edge_index.shape[1])

    atom_h = relu(_tconv(x, a_src, a_dst, eproj(edge_attr, a_mp, abg1), a_pad, abg1))
    edge_h = relu(_tconv(bag_x, b_src, b_dst, eproj(bag_edge_attr, b_mp, bag1), b_pad, bag1))
    atom_h = relu(_tconv(atom_h, a_src, a_dst, eproj(edge_h, a_mp, abg2), a_pad, abg2))
    edge_h = relu(_tconv(edge_h, b_src, b_dst, eproj(bag_edge_attr, b_mp, bag2), b_pad, bag2))
    atom_h = relu(_tconv(atom_h, a_src, a_dst, eproj(edge_h, a_mp, abg3), a_pad, abg3))
    edge_h = relu(_tconv(edge_h, b_src, b_dst, eproj(bag_edge_attr, b_mp, bag3), b_pad, bag3))
    ah = jax.ops.segment_max(atom_h, batch_ids, num_segments=B)
    ah = jnp.where(jnp.isfinite(ah), ah, 0.0)
    ah = _bn(ah @ abg_fc1_W + abg_fc1_b, abg_g1, abg_b1)
    ah = _bn(ah @ abg_fc2_W + abg_fc2_b, abg_g2, abg_b2)
    AA = relu(_tconv(tg_x, t_src, t_dst, eproj(tg_edge_attr, t_mp, tg1), t_pad, tg1))
    AA = relu(_tconv(AA, t_src, t_dst, eproj(tg_edge_attr, t_mp, tg2), t_pad, tg2))
    AA = relu(_tconv(AA, t_src, t_dst, eproj(tg_edge_attr, t_mp, tg3), t_pad, tg3))
    ssum = jax.ops.segment_sum(AA, tg_batch, num_segments=B)
    cnt = jax.ops.segment_sum(jnp.ones((AA.shape[0],), jnp.float32), tg_batch, num_segments=B)
    AA = ssum / jnp.maximum(cnt, 1.0)[:, None]
    AA = _bn(AA @ tg_fc1_W + tg_fc1_b, tg_g1, tg_b1)
    AA = _bn(AA @ tg_fc2_W + tg_fc2_b, tg_g2, tg_b2)
    o = jnp.concatenate([t_o, d_o, ah, AA], axis=-1)
    return _final_mm(o, Wout, bout)
